# baseline (speedup 1.0000x reference)
import functools

import jax
import jax.numpy as jnp
from jax import lax
from jax.experimental import pallas as pl
from jax.experimental.pallas import tpu as pltpu
from jax.experimental.pallas import tpu_sc as plsc

_NNZ = 4294967
_NC = 2
_NS = 16
_NW = _NC * _NS
_LANES = 16
_T = 16384
_NT = _NNZ // _T
_TAIL_OFF = _NT * _T
_TAIL = _NNZ - _TAIL_OFF
_TAIL_VECS = -(-_TAIL // _LANES)

_mesh = plsc.VectorSubcoreMesh(core_axis_name="c", subcore_axis_name="s",
                               num_cores=_NC, num_subcores=_NS)


def _mask_scale(vv, rv, i):
    sl = pl.ds(i * _LANES, _LANES)
    scale = jnp.where(rv[sl] + jnp.float32(0.5) >= jnp.float32(1.0),
                      jnp.float32(2.0), jnp.float32(0.0))
    vv[sl] = vv[sl] * scale


@functools.partial(
    pl.kernel,
    out_type=jax.ShapeDtypeStruct((_NNZ,), jnp.float32),
    mesh=_mesh,
    scratch_types=[
        pltpu.VMEM((_T,), jnp.float32),
        pltpu.VMEM((_T,), jnp.float32),
    ],
)
def _sparse_dropout_sc(vals_hbm, rand_hbm, out_hbm, vv, rv):
    wid = lax.axis_index("s") * _NC + lax.axis_index("c")
    lo = (wid * _NT) // _NW
    hi = ((wid + 1) * _NT) // _NW

    @pl.loop(lo, hi)
    def _tile(t):
        off = t * _T
        pltpu.sync_copy(vals_hbm.at[pl.ds(off, _T)], vv)
        pltpu.sync_copy(rand_hbm.at[pl.ds(off, _T)], rv)

        @plsc.parallel_loop(0, _T // _LANES, unroll=8)
        def _vec(i):
            _mask_scale(vv, rv, i)

        pltpu.sync_copy(vv, out_hbm.at[pl.ds(off, _T)])

    @pl.when(wid == 0)
    def _tail():
        pltpu.sync_copy(vals_hbm.at[pl.ds(_TAIL_OFF, _TAIL)],
                        vv.at[pl.ds(0, _TAIL)])
        pltpu.sync_copy(rand_hbm.at[pl.ds(_TAIL_OFF, _TAIL)],
                        rv.at[pl.ds(0, _TAIL)])

        @plsc.parallel_loop(0, _TAIL_VECS, unroll=4)
        def _vec(i):
            _mask_scale(vv, rv, i)

        pltpu.sync_copy(vv.at[pl.ds(0, _TAIL)],
                        out_hbm.at[pl.ds(_TAIL_OFF, _TAIL)])


def kernel(indices, values, rand_vals):
    return indices, _sparse_dropout_sc(values, rand_vals)


# speedup vs baseline: 1.3070x; 1.3070x over previous
import functools

import jax
import jax.numpy as jnp
from jax import lax
from jax.experimental import pallas as pl
from jax.experimental.pallas import tpu as pltpu
from jax.experimental.pallas import tpu_sc as plsc

_NNZ = 4294967
_NC = 2
_NS = 16
_NW = _NC * _NS
_LANES = 16
_TPW = 8
_T = 16768
_TAIL_OFF = _NW * _TPW * _T
_TAIL = _NNZ - _TAIL_OFF
_TAIL_VECS = -(-_TAIL // _LANES)

_mesh = plsc.VectorSubcoreMesh(core_axis_name="c", subcore_axis_name="s",
                               num_cores=_NC, num_subcores=_NS)


def _mask_scale(dst, vv, rv, i):
    sl = pl.ds(i * _LANES, _LANES)
    scale = jnp.where(rv[sl] + jnp.float32(0.5) >= jnp.float32(1.0),
                      jnp.float32(2.0), jnp.float32(0.0))
    dst[sl] = vv[sl] * scale


@functools.partial(
    pl.kernel,
    out_type=jax.ShapeDtypeStruct((_NNZ,), jnp.float32),
    mesh=_mesh,
    scratch_types=[
        pltpu.VMEM((2 * _T,), jnp.float32),
        pltpu.VMEM((2 * _T,), jnp.float32),
        pltpu.VMEM((2 * _T,), jnp.float32),
        pltpu.SemaphoreType.DMA,
        pltpu.SemaphoreType.DMA,
        pltpu.SemaphoreType.DMA,
    ],
)
def _sparse_dropout_sc(vals_hbm, rand_hbm, out_hbm, vv, rv, ov,
                       sem_v, sem_r, sem_o):
    wid = lax.axis_index("s") * _NC + lax.axis_index("c")
    base = wid * _TPW * _T

    def in_copies(t, b):
        off = base + t * _T
        cv = pltpu.make_async_copy(vals_hbm.at[pl.ds(off, _T)], vv.at[pl.ds(b * _T, _T)], sem_v)
        cr = pltpu.make_async_copy(rand_hbm.at[pl.ds(off, _T)], rv.at[pl.ds(b * _T, _T)], sem_r)
        return cv, cr

    def out_copy(t, b):
        off = base + t * _T
        return pltpu.make_async_copy(ov.at[pl.ds(b * _T, _T)], out_hbm.at[pl.ds(off, _T)], sem_o)

    for t in (0, 1):
        cv, cr = in_copies(t, t)
        cv.start()
        cr.start()

    for t in range(_TPW):
        b = t & 1
        cv, cr = in_copies(t, b)
        cv.wait()
        cr.wait()
        if t >= 2:
            out_copy(t - 2, b).wait()

        @plsc.parallel_loop(0, _T // _LANES, unroll=8)
        def _vec(i):
            _mask_scale(ov.at[pl.ds(b * _T, _T)], vv.at[pl.ds(b * _T, _T)], rv.at[pl.ds(b * _T, _T)], i)

        out_copy(t, b).start()
        if t + 2 < _TPW:
            nv, nr = in_copies(t + 2, b)
            nv.start()
            nr.start()

    out_copy(_TPW - 2, 0).wait()
    out_copy(_TPW - 1, 1).wait()

    @pl.when(wid == 0)
    def _tail():
        pltpu.sync_copy(vals_hbm.at[pl.ds(_TAIL_OFF, _TAIL)],
                        vv.at[pl.ds(0, _TAIL)])
        pltpu.sync_copy(rand_hbm.at[pl.ds(_TAIL_OFF, _TAIL)],
                        rv.at[pl.ds(0, _TAIL)])

        @plsc.parallel_loop(0, _TAIL_VECS, unroll=4)
        def _vec(i):
            _mask_scale(ov, vv, rv, i)

        pltpu.sync_copy(ov.at[pl.ds(0, _TAIL)],
                        out_hbm.at[pl.ds(_TAIL_OFF, _TAIL)])


def kernel(indices, values, rand_vals):
    return indices, _sparse_dropout_sc(values, rand_vals)


# speedup vs baseline: 1.4075x; 1.0768x over previous
import functools

import jax
import jax.numpy as jnp
from jax import lax
from jax.experimental import pallas as pl
from jax.experimental.pallas import tpu as pltpu
from jax.experimental.pallas import tpu_sc as plsc

_NNZ = 4294967
_NC = 2
_NS = 16
_NW = _NC * _NS
_LANES = 16
_TPW = 8
_T = 16768
_TAIL_OFF = _NW * _TPW * _T
_TAIL = _NNZ - _TAIL_OFF
_TAIL_VECS = -(-_TAIL // _LANES)

_mesh = plsc.VectorSubcoreMesh(core_axis_name="c", subcore_axis_name="s",
                               num_cores=_NC, num_subcores=_NS)


def _mask_scale(dst, vv, rv, i):
    sl = pl.ds(i * _LANES, _LANES)
    scale = jnp.where(rv[sl] + jnp.float32(0.5) >= jnp.float32(1.0),
                      jnp.float32(2.0), jnp.float32(0.0))
    dst[sl] = vv[sl] * scale


@functools.partial(
    pl.kernel,
    out_type=jax.ShapeDtypeStruct((_NNZ,), jnp.float32),
    mesh=_mesh,
    scratch_types=[
        pltpu.VMEM((2 * _T,), jnp.float32),
        pltpu.VMEM((2 * _T,), jnp.float32),
        pltpu.VMEM((2 * _T,), jnp.float32),
        pltpu.SemaphoreType.DMA,
        pltpu.SemaphoreType.DMA,
        pltpu.SemaphoreType.DMA,
    ],
)
def _sparse_dropout_sc(vals_hbm, rand_hbm, out_hbm, vv, rv, ov,
                       sem_v, sem_r, sem_o):
    wid = lax.axis_index("s") * _NC + lax.axis_index("c")
    base = wid * _TPW * _T

    def in_copies(t, b):
        off = base + t * _T
        cv = pltpu.make_async_copy(vals_hbm.at[pl.ds(off, _T)], vv.at[pl.ds(b * _T, _T)], sem_v)
        cr = pltpu.make_async_copy(rand_hbm.at[pl.ds(off, _T)], rv.at[pl.ds(b * _T, _T)], sem_r)
        return cv, cr

    def out_copy(t, b):
        off = base + t * _T
        return pltpu.make_async_copy(ov.at[pl.ds(b * _T, _T)], out_hbm.at[pl.ds(off, _T)], sem_o)

    for t in (0, 1):
        cv, cr = in_copies(t, t)
        cv.start()
        cr.start()

    for t in range(_TPW):
        b = t & 1
        cv, cr = in_copies(t, b)
        cv.wait()
        cr.wait()
        if t >= 2:
            out_copy(t - 2, b).wait()

        @plsc.parallel_loop(0, _T // _LANES, unroll=8)
        def _vec(i):
            _mask_scale(ov.at[pl.ds(b * _T, _T)], vv.at[pl.ds(b * _T, _T)], rv.at[pl.ds(b * _T, _T)], i)

        out_copy(t, b).start()
        if t + 2 < _TPW:
            nv, nr = in_copies(t + 2, b)
            nv.start()
            nr.start()

    out_copy(_TPW - 2, 0).wait()
    out_copy(_TPW - 1, 1).wait()

    @pl.when(wid == 0)
    def _tail():
        pltpu.sync_copy(vals_hbm.at[pl.ds(_TAIL_OFF, _TAIL)],
                        vv.at[pl.ds(0, _TAIL)])
        pltpu.sync_copy(rand_hbm.at[pl.ds(_TAIL_OFF, _TAIL)],
                        rv.at[pl.ds(0, _TAIL)])

        @plsc.parallel_loop(0, _TAIL_VECS, unroll=4)
        def _vec(i):
            _mask_scale(ov, vv, rv, i)

        pltpu.sync_copy(ov.at[pl.ds(0, _TAIL)],
                        out_hbm.at[pl.ds(_TAIL_OFF, _TAIL)])


_CB = 262144
_CGRID = -(-_NNZ // _CB)


def _copy_body(src_ref, dst_ref):
    dst_ref[...] = src_ref[...]


def _indices_copy_tc(indices):
    return pl.pallas_call(
        _copy_body,
        out_shape=jax.ShapeDtypeStruct((2, _NNZ), jnp.int32),
        grid=(_CGRID,),
        in_specs=[pl.BlockSpec((2, _CB), lambda i: (0, i))],
        out_specs=pl.BlockSpec((2, _CB), lambda i: (0, i)),
    )(indices)


def kernel(indices, values, rand_vals):
    return _indices_copy_tc(indices), _sparse_dropout_sc(values, rand_vals)


# speedup vs baseline: 1.4165x; 1.0064x over previous
import functools

import jax
import jax.numpy as jnp
from jax import lax
from jax.experimental import pallas as pl
from jax.experimental.pallas import tpu as pltpu
from jax.experimental.pallas import tpu_sc as plsc

_NNZ = 4294967
_NC = 2
_NS = 16
_NW = _NC * _NS
_LANES = 16
_TPW = 8
_T = 16768
_TAIL_OFF = _NW * _TPW * _T
_TAIL = _NNZ - _TAIL_OFF
_TAIL_VECS = -(-_TAIL // _LANES)

_mesh = plsc.VectorSubcoreMesh(core_axis_name="c", subcore_axis_name="s",
                               num_cores=_NC, num_subcores=_NS)


def _mask_scale(dst, vv, rv, i):
    sl = pl.ds(i * _LANES, _LANES)
    scale = jnp.where(rv[sl] + jnp.float32(0.5) >= jnp.float32(1.0),
                      jnp.float32(2.0), jnp.float32(0.0))
    dst[sl] = vv[sl] * scale


@functools.partial(
    pl.kernel,
    out_type=jax.ShapeDtypeStruct((_NNZ,), jnp.float32),
    mesh=_mesh,
    scratch_types=[
        pltpu.VMEM((2 * _T,), jnp.float32),
        pltpu.VMEM((2 * _T,), jnp.float32),
        pltpu.VMEM((2 * _T,), jnp.float32),
        pltpu.SemaphoreType.DMA,
        pltpu.SemaphoreType.DMA,
        pltpu.SemaphoreType.DMA,
    ],
)
def _sparse_dropout_sc(vals_hbm, rand_hbm, out_hbm, vv, rv, ov,
                       sem_v, sem_r, sem_o):
    wid = lax.axis_index("s") * _NC + lax.axis_index("c")
    base = wid * _TPW * _T

    def in_copies(t, b):
        off = base + t * _T
        cv = pltpu.make_async_copy(vals_hbm.at[pl.ds(off, _T)], vv.at[pl.ds(b * _T, _T)], sem_v)
        cr = pltpu.make_async_copy(rand_hbm.at[pl.ds(off, _T)], rv.at[pl.ds(b * _T, _T)], sem_r)
        return cv, cr

    def out_copy(t, b):
        off = base + t * _T
        return pltpu.make_async_copy(ov.at[pl.ds(b * _T, _T)], out_hbm.at[pl.ds(off, _T)], sem_o)

    def compute(b):
        @plsc.parallel_loop(0, _T // _LANES, unroll=8)
        def _vec(i):
            _mask_scale(ov.at[pl.ds(b * _T, _T)], vv.at[pl.ds(b * _T, _T)],
                        rv.at[pl.ds(b * _T, _T)], i)

    for t in (0, 1):
        cv, cr = in_copies(t, t)
        cv.start()
        cr.start()

    @pl.loop(0, _TPW // 2)
    def _pair(p):
        t0 = p * 2
        for b in (0, 1):
            t = t0 + b
            cv, cr = in_copies(t, b)
            cv.wait()
            cr.wait()

            @pl.when(t >= 2)
            def _():
                out_copy(t - 2, b).wait()

            compute(b)
            out_copy(t, b).start()

            @pl.when(t + 2 < _TPW)
            def _():
                nv, nr = in_copies(t + 2, b)
                nv.start()
                nr.start()

    out_copy(_TPW - 2, 0).wait()
    out_copy(_TPW - 1, 1).wait()

    @pl.when(wid == 0)
    def _tail():
        pltpu.sync_copy(vals_hbm.at[pl.ds(_TAIL_OFF, _TAIL)],
                        vv.at[pl.ds(0, _TAIL)])
        pltpu.sync_copy(rand_hbm.at[pl.ds(_TAIL_OFF, _TAIL)],
                        rv.at[pl.ds(0, _TAIL)])

        @plsc.parallel_loop(0, _TAIL_VECS, unroll=4)
        def _vec(i):
            _mask_scale(ov, vv, rv, i)

        pltpu.sync_copy(ov.at[pl.ds(0, _TAIL)],
                        out_hbm.at[pl.ds(_TAIL_OFF, _TAIL)])


_CB = 262144
_CGRID = -(-_NNZ // _CB)


def _copy_body(src_ref, dst_ref):
    dst_ref[...] = src_ref[...]


def _indices_copy_tc(indices):
    return pl.pallas_call(
        _copy_body,
        out_shape=jax.ShapeDtypeStruct((2, _NNZ), jnp.int32),
        grid=(_CGRID,),
        in_specs=[pl.BlockSpec((2, _CB), lambda i: (0, i))],
        out_specs=pl.BlockSpec((2, _CB), lambda i: (0, i)),
    )(indices)


def kernel(indices, values, rand_vals):
    return _indices_copy_tc(indices), _sparse_dropout_sc(values, rand_vals)


# speedup vs baseline: 1.4267x; 1.0072x over previous
import functools

import jax
import jax.numpy as jnp
from jax import lax
from jax.experimental import pallas as pl
from jax.experimental.pallas import tpu as pltpu
from jax.experimental.pallas import tpu_sc as plsc

_NNZ = 4294967
_NC = 2
_NS = 16
_NW = _NC * _NS
_LANES = 16
_TPW = 8
_T = 16768
_TAIL_OFF = _NW * _TPW * _T
_TAIL = _NNZ - _TAIL_OFF
_TAIL_VECS = -(-_TAIL // _LANES)

_mesh = plsc.VectorSubcoreMesh(core_axis_name="c", subcore_axis_name="s",
                               num_cores=_NC, num_subcores=_NS)


def _mask_scale(dst, vv, rv, i):
    sl = pl.ds(i * _LANES, _LANES)
    scale = jnp.where(rv[sl] + jnp.float32(0.5) >= jnp.float32(1.0),
                      jnp.float32(2.0), jnp.float32(0.0))
    dst[sl] = vv[sl] * scale


@functools.partial(
    pl.kernel,
    out_type=jax.ShapeDtypeStruct((_NNZ,), jnp.float32),
    mesh=_mesh,
    scratch_types=[
        pltpu.VMEM((2 * _T,), jnp.float32),
        pltpu.VMEM((2 * _T,), jnp.float32),
        pltpu.VMEM((2 * _T,), jnp.float32),
        pltpu.SemaphoreType.DMA,
        pltpu.SemaphoreType.DMA,
        pltpu.SemaphoreType.DMA,
    ],
)
def _sparse_dropout_sc(vals_hbm, rand_hbm, out_hbm, vv, rv, ov,
                       sem_v, sem_r, sem_o):
    wid = lax.axis_index("s") * _NC + lax.axis_index("c")
    base = wid * _TPW * _T

    def in_copies(t, b):
        off = base + t * _T
        cv = pltpu.make_async_copy(vals_hbm.at[pl.ds(off, _T)], vv.at[pl.ds(b * _T, _T)], sem_v)
        cr = pltpu.make_async_copy(rand_hbm.at[pl.ds(off, _T)], rv.at[pl.ds(b * _T, _T)], sem_r)
        return cv, cr

    def out_copy(t, b):
        off = base + t * _T
        return pltpu.make_async_copy(ov.at[pl.ds(b * _T, _T)], out_hbm.at[pl.ds(off, _T)], sem_o)

    def compute(b):
        @plsc.parallel_loop(0, _T // _LANES, unroll=8)
        def _vec(i):
            _mask_scale(ov.at[pl.ds(b * _T, _T)], vv.at[pl.ds(b * _T, _T)],
                        rv.at[pl.ds(b * _T, _T)], i)

    for t in (0, 1):
        cv, cr = in_copies(t, t)
        cv.start()
        cr.start()

    @pl.loop(0, _TPW // 2)
    def _pair(p):
        t0 = p * 2
        for b in (0, 1):
            t = t0 + b
            cv, cr = in_copies(t, b)
            cv.wait()
            cr.wait()

            @pl.when(t >= 2)
            def _():
                out_copy(t - 2, b).wait()

            compute(b)
            out_copy(t, b).start()

            @pl.when(t + 2 < _TPW)
            def _():
                nv, nr = in_copies(t + 2, b)
                nv.start()
                nr.start()

    out_copy(_TPW - 2, 0).wait()
    out_copy(_TPW - 1, 1).wait()

    @pl.when(wid == 0)
    def _tail():
        pltpu.sync_copy(vals_hbm.at[pl.ds(_TAIL_OFF, _TAIL)],
                        vv.at[pl.ds(0, _TAIL)])
        pltpu.sync_copy(rand_hbm.at[pl.ds(_TAIL_OFF, _TAIL)],
                        rv.at[pl.ds(0, _TAIL)])

        @plsc.parallel_loop(0, _TAIL_VECS, unroll=4)
        def _vec(i):
            _mask_scale(ov, vv, rv, i)

        pltpu.sync_copy(ov.at[pl.ds(0, _TAIL)],
                        out_hbm.at[pl.ds(_TAIL_OFF, _TAIL)])


_CB = 1048576
_CGRID = -(-_NNZ // _CB)


def _copy_body(src_ref, dst_ref):
    dst_ref[...] = src_ref[...]


def _indices_copy_tc(indices):
    return pl.pallas_call(
        _copy_body,
        out_shape=jax.ShapeDtypeStruct((2, _NNZ), jnp.int32),
        grid=(_CGRID,),
        in_specs=[pl.BlockSpec((2, _CB), lambda i: (0, i))],
        out_specs=pl.BlockSpec((2, _CB), lambda i: (0, i)),
    )(indices)


def kernel(indices, values, rand_vals):
    return _indices_copy_tc(indices), _sparse_dropout_sc(values, rand_vals)


# speedup vs baseline: 1.4303x; 1.0025x over previous
import functools

import jax
import jax.numpy as jnp
from jax import lax
from jax.experimental import pallas as pl
from jax.experimental.pallas import tpu as pltpu
from jax.experimental.pallas import tpu_sc as plsc

_NNZ = 4294967
_NC = 2
_NS = 16
_NW = _NC * _NS
_LANES = 16
_TPW = 8
_T = 16768
_TAIL_OFF = _NW * _TPW * _T
_TAIL = _NNZ - _TAIL_OFF
_TAIL_VECS = -(-_TAIL // _LANES)

def _mask_scale(dst, vv, rv, i):
    sl = pl.ds(i * _LANES, _LANES)
    scale = jnp.where(rv[sl] + jnp.float32(0.5) >= jnp.float32(1.0),
                      jnp.float32(2.0), jnp.float32(0.0))
    dst[sl] = vv[sl] * scale


@functools.cache
def _build_sparse_dropout_sc():
    mesh = plsc.VectorSubcoreMesh(core_axis_name="c", subcore_axis_name="s",
                                  num_cores=_NC, num_subcores=_NS)
    return functools.partial(
        pl.kernel,
        out_type=jax.ShapeDtypeStruct((_NNZ,), jnp.float32),
        mesh=mesh,
        scratch_types=[
            pltpu.VMEM((2 * _T,), jnp.float32),
            pltpu.VMEM((2 * _T,), jnp.float32),
            pltpu.VMEM((2 * _T,), jnp.float32),
            pltpu.SemaphoreType.DMA,
            pltpu.SemaphoreType.DMA,
            pltpu.SemaphoreType.DMA,
        ],
    )(_sparse_dropout_sc_body)


def _sparse_dropout_sc_body(vals_hbm, rand_hbm, out_hbm, vv, rv, ov,
                            sem_v, sem_r, sem_o):
    wid = lax.axis_index("s") * _NC + lax.axis_index("c")
    base = wid * _TPW * _T

    def in_copies(t, b):
        off = base + t * _T
        cv = pltpu.make_async_copy(vals_hbm.at[pl.ds(off, _T)], vv.at[pl.ds(b * _T, _T)], sem_v)
        cr = pltpu.make_async_copy(rand_hbm.at[pl.ds(off, _T)], rv.at[pl.ds(b * _T, _T)], sem_r)
        return cv, cr

    def out_copy(t, b):
        off = base + t * _T
        return pltpu.make_async_copy(ov.at[pl.ds(b * _T, _T)], out_hbm.at[pl.ds(off, _T)], sem_o)

    def compute(b):
        @plsc.parallel_loop(0, _T // _LANES, unroll=8)
        def _vec(i):
            _mask_scale(ov.at[pl.ds(b * _T, _T)], vv.at[pl.ds(b * _T, _T)],
                        rv.at[pl.ds(b * _T, _T)], i)

    for t in (0, 1):
        cv, cr = in_copies(t, t)
        cv.start()
        cr.start()

    @pl.loop(0, _TPW // 2)
    def _pair(p):
        t0 = p * 2
        for b in (0, 1):
            t = t0 + b
            cv, cr = in_copies(t, b)
            cv.wait()
            cr.wait()

            @pl.when(t >= 2)
            def _():
                out_copy(t - 2, b).wait()

            compute(b)
            out_copy(t, b).start()

            @pl.when(t + 2 < _TPW)
            def _():
                nv, nr = in_copies(t + 2, b)
                nv.start()
                nr.start()

    out_copy(_TPW - 2, 0).wait()
    out_copy(_TPW - 1, 1).wait()

    @pl.when(wid == 0)
    def _tail():
        pltpu.sync_copy(vals_hbm.at[pl.ds(_TAIL_OFF, _TAIL)],
                        vv.at[pl.ds(0, _TAIL)])
        pltpu.sync_copy(rand_hbm.at[pl.ds(_TAIL_OFF, _TAIL)],
                        rv.at[pl.ds(0, _TAIL)])

        @plsc.parallel_loop(0, _TAIL_VECS, unroll=4)
        def _vec(i):
            _mask_scale(ov, vv, rv, i)

        pltpu.sync_copy(ov.at[pl.ds(0, _TAIL)],
                        out_hbm.at[pl.ds(_TAIL_OFF, _TAIL)])


_CB = 1048576
_CGRID = -(-_NNZ // _CB)


def _copy_body(src_ref, dst_ref):
    dst_ref[...] = src_ref[...]


def _indices_copy_tc(indices):
    return pl.pallas_call(
        _copy_body,
        out_shape=jax.ShapeDtypeStruct((2, _NNZ), jnp.int32),
        grid=(_CGRID,),
        in_specs=[pl.BlockSpec((2, _CB), lambda i: (0, i))],
        out_specs=pl.BlockSpec((2, _CB), lambda i: (0, i)),
    )(indices)


def kernel(indices, values, rand_vals):
    new_vals = _build_sparse_dropout_sc()(values, rand_vals)
    return _indices_copy_tc(indices), new_vals
